# baseline (speedup 1.0000x reference)
import functools
import math

import jax
import jax.numpy as jnp
from jax import lax
from jax.experimental import pallas as pl
from jax.experimental.pallas import tpu as pltpu
from jax.experimental.pallas import tpu_sc as plsc

NC = 2
NS = 16
NW = NC * NS
CH = 128
KPC = 4
UNIT = CH * KPC


def _make_lookup(T, Btok, V, D, scale):
    n_bc = Btok // UNIT
    n_units = T * n_bc
    upw = n_units // NW
    n_dblk = D // 8
    ntile_b = Btok // CH
    strip = KPC * 8 * CH
    out_len = Btok * T * D
    mesh = plsc.VectorSubcoreMesh(
        core_axis_name="c", subcore_axis_name="s",
        num_cores=NC, num_subcores=NS)

    @functools.partial(
        pl.kernel,
        out_type=jax.ShapeDtypeStruct((out_len,), jnp.float32),
        mesh=mesh,
        scratch_types=[
            pltpu.VMEM((2, KPC, CH), jnp.int32),
            pltpu.VMEM((KPC, CH, D), jnp.float32),
            pltpu.VMEM((2, UNIT * D), jnp.float32),
            pltpu.VMEM((D // 16, 16), jnp.int32),
            pltpu.SemaphoreType.DMA((KPC,)),
            pltpu.SemaphoreType.DMA((2,)),
        ],
        compiler_params=pltpu.CompilerParams(
            use_tc_tiling_on_sc=False, needs_layout_passes=False),
    )
    def lookup(tok4, pos_hbm, table_hbm, out_hbm,
               idx_v, rows_v, tbuf, pos_v, gsem, osem):
        wid = lax.axis_index("s") * NC + lax.axis_index("c")
        u0 = wid * upw

        pltpu.sync_copy(pos_hbm, pos_v)
        pos_c = [pos_v[c, :] for c in range(D // 16)]

        def stage_idx(u, s):
            pltpu.sync_copy(tok4.at[u // n_bc, u % n_bc], idx_v.at[s])

        def gather(s, k):
            return pltpu.make_async_copy(
                table_hbm.at[idx_v.at[s, k]], rows_v.at[k], gsem.at[k])

        def out_copy(u, nb, db):
            t = u // n_bc
            bc = u % n_bc
            off = ((t * n_dblk + db) * ntile_b + bc * KPC) * (8 * CH)
            return pltpu.make_async_copy(
                tbuf.at[nb, pl.ds(db * strip, strip)],
                out_hbm.at[pl.ds(off, strip)], osem.at[nb])

        def do_unit(r, nb):
            u = u0 + r
            s, s_next = nb, 1 - nb

            @pl.when(r + 1 < upw)
            def _():
                stage_idx(u + 1, s_next)

            @pl.when(r >= 2)
            def _():
                for db in range(n_dblk):
                    out_copy(u - 2, nb, db).wait()

            for k in range(KPC):
                gather(s, k).wait()

                @pl.loop(0, CH, unroll=4)
                def _row(rr):
                    sofs = k * (8 * CH) + rr
                    for c in range(D // 16):
                        v = rows_v[k, rr, pl.ds(c * 16, 16)] * scale
                        plsc.store_scatter(
                            tbuf.at[nb], [pos_c[c] + sofs], v)

                @pl.when(r + 1 < upw)
                def _():
                    gather(s_next, k).start()

            for db in range(n_dblk):
                out_copy(u, nb, db).start()

        stage_idx(u0, 0)
        for k in range(KPC):
            gather(0, k).start()

        @pl.loop(0, upw // 2)
        def _pair(step):
            do_unit(step * 2, 0)
            do_unit(step * 2 + 1, 1)

        for r in (upw - 2, upw - 1):
            for db in range(n_dblk):
                out_copy(u0 + r, r % 2, db).wait()

    return lookup


def kernel(tokens, table):
    Btok, T = tokens.shape
    V, D = table.shape
    assert (Btok * T) % (NW * UNIT) == 0 and D % 16 == 0
    assert ((Btok * T) // (NW * UNIT)) % 2 == 0
    scale = math.sqrt(D)

    strip = KPC * 8 * CH
    pos = jnp.asarray(
        [[(d // 8) * strip + (d % 8) * CH for d in range(c * 16, c * 16 + 16)]
         for c in range(D // 16)], dtype=jnp.int32)
    tok4 = tokens.T.astype(jnp.int32).reshape(T, Btok // UNIT, KPC, CH)
    outf = _make_lookup(T, Btok, V, D, scale)(tok4, pos, table)
    out5 = outf.reshape(T, D // 8, Btok // CH, 8, CH)
    return out5.transpose(2, 4, 0, 1, 3).reshape(Btok, T, D)


# speedup vs baseline: 1.3048x; 1.3048x over previous
import functools
import math

import jax
import jax.numpy as jnp
from jax import lax
from jax.experimental import pallas as pl
from jax.experimental.pallas import tpu as pltpu
from jax.experimental.pallas import tpu_sc as plsc

NC = 2
NS = 16
NW = NC * NS
CH = 128
KPC = 4
UNIT = CH * KPC


def _make_lookup(T, Btok, V, D, scale):
    n_bc = Btok // UNIT
    n_units = T * n_bc
    upw = n_units // NW
    n_dblk = D // 8
    ntile_b = Btok // CH
    strip = KPC * 8 * CH
    out_len = Btok * T * D
    mesh = plsc.VectorSubcoreMesh(
        core_axis_name="c", subcore_axis_name="s",
        num_cores=NC, num_subcores=NS)

    @functools.partial(
        pl.kernel,
        out_type=jax.ShapeDtypeStruct((out_len,), jnp.float32),
        mesh=mesh,
        scratch_types=[
            pltpu.VMEM((2, KPC, CH), jnp.int32),
            pltpu.VMEM((KPC, CH, D), jnp.float32),
            pltpu.VMEM((2, UNIT * D), jnp.float32),
            pltpu.VMEM((D // 16, 16), jnp.int32),
            pltpu.SemaphoreType.DMA((KPC,)),
            pltpu.SemaphoreType.DMA((2,)),
        ],
        compiler_params=pltpu.CompilerParams(
            use_tc_tiling_on_sc=False, needs_layout_passes=False),
    )
    def lookup(tok4, pos_hbm, table_hbm, out_hbm,
               idx_v, rows_v, tbuf, pos_v, gsem, osem):
        wid = lax.axis_index("s") * NC + lax.axis_index("c")
        u0 = wid * upw

        pltpu.sync_copy(pos_hbm, pos_v)
        pos_c = [pos_v[c, :] for c in range(D // 16)]

        def stage_idx(u, s):
            pltpu.sync_copy(tok4.at[u // n_bc, u % n_bc], idx_v.at[s])

        def gather(s, k):
            return pltpu.make_async_copy(
                table_hbm.at[idx_v.at[s, k]], rows_v.at[k], gsem.at[k])

        def out_copy(u, nb, db):
            t = u // n_bc
            bc = u % n_bc
            off = ((t * n_dblk + db) * ntile_b + bc * KPC) * (8 * CH)
            return pltpu.make_async_copy(
                tbuf.at[nb, pl.ds(db * strip, strip)],
                out_hbm.at[pl.ds(off, strip)], osem.at[nb])

        def do_unit(r, nb):
            u = u0 + r
            s, s_next = nb, 1 - nb

            @pl.when(r + 1 < upw)
            def _():
                stage_idx(u + 1, s_next)

            @pl.when(r >= 2)
            def _():
                for db in range(n_dblk):
                    out_copy(u - 2, nb, db).wait()

            for k in range(KPC):
                gather(s, k).wait()
                posk_c = [p + (k * 8 * CH) for p in pos_c]

                @plsc.parallel_loop(0, CH, unroll=8)
                def _row(rr):
                    for c in range(D // 16):
                        v = rows_v[k, rr, pl.ds(c * 16, 16)] * scale
                        plsc.store_scatter(
                            tbuf.at[nb], [posk_c[c] + rr], v)

                @pl.when(r + 1 < upw)
                def _():
                    gather(s_next, k).start()

            for db in range(n_dblk):
                out_copy(u, nb, db).start()

        stage_idx(u0, 0)
        for k in range(KPC):
            gather(0, k).start()

        @pl.loop(0, upw // 2)
        def _pair(step):
            do_unit(step * 2, 0)
            do_unit(step * 2 + 1, 1)

        for r in (upw - 2, upw - 1):
            for db in range(n_dblk):
                out_copy(u0 + r, r % 2, db).wait()

    return lookup


def kernel(tokens, table):
    Btok, T = tokens.shape
    V, D = table.shape
    assert (Btok * T) % (NW * UNIT) == 0 and D % 16 == 0
    assert ((Btok * T) // (NW * UNIT)) % 2 == 0
    scale = math.sqrt(D)

    strip = KPC * 8 * CH
    pos = jnp.asarray(
        [[(d // 8) * strip + (d % 8) * CH for d in range(c * 16, c * 16 + 16)]
         for c in range(D // 16)], dtype=jnp.int32)
    tok4 = tokens.T.astype(jnp.int32).reshape(T, Btok // UNIT, KPC, CH)
    outf = _make_lookup(T, Btok, V, D, scale)(tok4, pos, table)
    out5 = outf.reshape(T, D // 8, Btok // CH, 8, CH)
    return out5.transpose(2, 4, 0, 1, 3).reshape(Btok, T, D)


# speedup vs baseline: 1.4558x; 1.1158x over previous
import functools
import math

import jax
import jax.numpy as jnp
from jax import lax
from jax.experimental import pallas as pl
from jax.experimental.pallas import tpu as pltpu
from jax.experimental.pallas import tpu_sc as plsc

NC = 2
NS = 16
NW = NC * NS
CH = 128
NR = 8


def _make_lookup(T, Btok, V, D, scale):
    n_ch = T * (Btok // CH) // NW
    assert Btok == NW * CH
    assert n_ch % NR == 0
    mesh = plsc.VectorSubcoreMesh(
        core_axis_name="c", subcore_axis_name="s",
        num_cores=NC, num_subcores=NS)

    @functools.partial(
        pl.kernel,
        out_type=jax.ShapeDtypeStruct((T, Btok, D), jnp.float32),
        mesh=mesh,
        scratch_types=[
            pltpu.VMEM((T, CH), jnp.int32),
            pltpu.VMEM((NR, CH, D), jnp.float32),
            pltpu.SemaphoreType.DMA((NR,)),
            pltpu.SemaphoreType.DMA((NR,)),
        ],
        compiler_params=pltpu.CompilerParams(use_tc_tiling_on_sc=False),
    )
    def lookup(tokR, table_hbm, out_hbm, idx_v, rows_v, gsem, osem):
        wid = lax.axis_index("s") * NC + lax.axis_index("c")
        b0 = wid * CH

        pltpu.sync_copy(tokR.at[wid], idx_v)

        def gather(t, rb):
            return pltpu.make_async_copy(
                table_hbm.at[idx_v.at[t]], rows_v.at[rb], gsem.at[rb])

        def out_copy(t, rb):
            return pltpu.make_async_copy(
                rows_v.at[rb], out_hbm.at[t, pl.ds(b0, CH)], osem.at[rb])

        for rb in range(NR):
            gather(rb, rb).start()

        def round_body(step, refill):
            for rb in range(NR):
                t = step * NR + rb
                gather(t, rb).wait()

                @plsc.parallel_loop(0, CH, unroll=8)
                def _scale(rr):
                    for c in range(D // 16):
                        sl = pl.ds(c * 16, 16)
                        rows_v[rb, rr, sl] = rows_v[rb, rr, sl] * scale

                cp = out_copy(t, rb)
                cp.start()
                cp.wait()
                if refill:
                    gather(t + NR, rb).start()

        n_outer = n_ch // NR

        @pl.loop(0, n_outer - 1)
        def _main(step):
            round_body(step, refill=True)

        round_body(n_outer - 1, refill=False)

    return lookup


def kernel(tokens, table):
    Btok, T = tokens.shape
    V, D = table.shape
    assert Btok == NW * CH and D % 16 == 0
    scale = math.sqrt(D)

    tokR = tokens.T.astype(jnp.int32).reshape(T, NW, CH).transpose(1, 0, 2)
    out3 = _make_lookup(T, Btok, V, D, scale)(tokR, table)
    return out3.transpose(1, 0, 2)
